# baseline (speedup 1.0000x reference)
import functools
import math

import jax
import jax.numpy as jnp
from jax.experimental import pallas as pl
from jax.experimental.pallas import tpu as pltpu


def _sortkey(x):
    i = jax.lax.bitcast_convert_type(x, jnp.int32)
    key = i ^ ((i >> 31) & jnp.int32(0x7FFFFFFF))
    return jax.lax.bitcast_convert_type(key, jnp.uint32) ^ jnp.uint32(0x80000000)


def _proj_kernel(x_ref, w1_ref, b1_ref, w2_ref, b2_ref, q_ref, k_ref):
    x = x_ref[0]
    dn = (((1,), (1,)), ((), ()))
    q_ref[0] = (
        jax.lax.dot_general(x, w1_ref[...], dn, preferred_element_type=jnp.float32)
        + b1_ref[...]
    )
    k_ref[0] = (
        jax.lax.dot_general(x, w2_ref[...], dn, preferred_element_type=jnp.float32)
        + b2_ref[...]
    )


def _thresh_kernel(kk, q_ref, kall_ref, t_ref, j_ref):
    q = q_ref[0]
    ka = kall_ref[0]
    dn = (((1,), (1,)), ((), ()))
    s = jax.lax.dot_general(q, ka, dn, preferred_element_type=jnp.float32)
    u = _sortkey(s)
    rb, n = s.shape

    def bit_body(i, t):
        bit = (jnp.int32(31) - i).astype(jnp.uint32)
        cand = t | jax.lax.shift_left(jnp.uint32(1), bit)
        cnt = jnp.sum((u >= cand).astype(jnp.int32), axis=1, keepdims=True)
        return jnp.where(cnt >= kk, cand, t)

    t = jax.lax.fori_loop(0, 32, bit_body, jnp.zeros((rb, 1), jnp.uint32))

    cnt_gt = jnp.sum((u > t).astype(jnp.int32), axis=1, keepdims=True)
    need = kk - cnt_gt
    eq = u == t
    col = jax.lax.broadcasted_iota(jnp.int32, (rb, n), 1)

    def j_body(i, carry):
        lo, hi = carry
        mid = (lo + hi) >> 1
        cnt = jnp.sum((eq & (col <= mid)).astype(jnp.int32), axis=1, keepdims=True)
        p = cnt >= need
        return jnp.where(p, lo, mid + 1), jnp.where(p, mid, hi)

    lo, _ = jax.lax.fori_loop(
        0,
        max(1, (n - 1).bit_length()),
        j_body,
        (jnp.zeros((rb, 1), jnp.int32), jnp.full((rb, 1), n - 1, jnp.int32)),
    )
    t_ref[0] = t.reshape(1, rb)
    j_ref[0] = lo.reshape(1, rb)


def _mask_kernel(rb, qi_ref, ki_ref, qall_ref, kall_ref, trow_ref, jrow_ref,
                 tcol_ref, jcol_ref, sc_ref, out_ref):
    qi = qi_ref[0]
    ki = ki_ref[0]
    qa = qall_ref[0]
    ka = kall_ref[0]
    dn = (((1,), (1,)), ((), ()))
    s1 = jax.lax.dot_general(qi, ka, dn, preferred_element_type=jnp.float32)
    s2 = jax.lax.dot_general(ki, qa, dn, preferred_element_type=jnp.float32)
    u1 = _sortkey(s1)
    u2 = _sortkey(s2)
    n = s1.shape[1]

    t1 = trow_ref[0].reshape(rb, 1)
    j1 = jrow_ref[0].reshape(rb, 1)
    t2 = tcol_ref[0]
    j2 = jcol_ref[0]

    col = jax.lax.broadcasted_iota(jnp.int32, (rb, n), 1)
    row = jax.lax.broadcasted_iota(jnp.int32, (rb, n), 0) + pl.program_id(1) * rb

    m1 = (u1 > t1) | ((u1 == t1) & (col <= j1))
    m2 = (u2 > t2) | ((u2 == t2) & (row <= j2))
    out_ref[0] = (m1.astype(jnp.float32) + m2.astype(jnp.float32)) * sc_ref[0, 0]


def kernel(src_temp, W1, b1, W2, b2, k):
    B, N, d_model = src_temp.shape
    d_ff = W1.shape[0]
    kk = d_ff

    rb = 256
    while N % rb:
        rb //= 2
    nb = N // rb

    b1r = b1.reshape(1, d_ff)
    b2r = b2.reshape(1, d_ff)

    q, kmat = pl.pallas_call(
        _proj_kernel,
        grid=(B,),
        in_specs=[
            pl.BlockSpec((1, N, d_model), lambda b: (b, 0, 0)),
            pl.BlockSpec((d_ff, d_model), lambda b: (0, 0)),
            pl.BlockSpec((1, d_ff), lambda b: (0, 0)),
            pl.BlockSpec((d_ff, d_model), lambda b: (0, 0)),
            pl.BlockSpec((1, d_ff), lambda b: (0, 0)),
        ],
        out_specs=[
            pl.BlockSpec((1, N, d_ff), lambda b: (b, 0, 0)),
            pl.BlockSpec((1, N, d_ff), lambda b: (b, 0, 0)),
        ],
        out_shape=[
            jax.ShapeDtypeStruct((B, N, d_ff), jnp.float32),
            jax.ShapeDtypeStruct((B, N, d_ff), jnp.float32),
        ],
        compiler_params=pltpu.CompilerParams(dimension_semantics=("parallel",)),
    )(src_temp, W1, b1r, W2, b2r)

    thr, jcut = pl.pallas_call(
        functools.partial(_thresh_kernel, kk),
        grid=(B, nb),
        in_specs=[
            pl.BlockSpec((1, rb, d_ff), lambda b, i: (b, i, 0)),
            pl.BlockSpec((1, N, d_ff), lambda b, i: (b, 0, 0)),
        ],
        out_specs=[
            pl.BlockSpec((1, 1, rb), lambda b, i: (b * nb + i, 0, 0)),
            pl.BlockSpec((1, 1, rb), lambda b, i: (b * nb + i, 0, 0)),
        ],
        out_shape=[
            jax.ShapeDtypeStruct((B * nb, 1, rb), jnp.uint32),
            jax.ShapeDtypeStruct((B * nb, 1, rb), jnp.int32),
        ],
        compiler_params=pltpu.CompilerParams(
            dimension_semantics=("parallel", "parallel")
        ),
    )(q, kmat)

    thr_col = thr.reshape(B, 1, N)
    jcut_col = jcut.reshape(B, 1, N)
    scale = (jnp.asarray(k, jnp.float32) * (0.5 / d_ff)).reshape(1, 1)

    out = pl.pallas_call(
        functools.partial(_mask_kernel, rb),
        grid=(B, nb),
        in_specs=[
            pl.BlockSpec((1, rb, d_ff), lambda b, i: (b, i, 0)),
            pl.BlockSpec((1, rb, d_ff), lambda b, i: (b, i, 0)),
            pl.BlockSpec((1, N, d_ff), lambda b, i: (b, 0, 0)),
            pl.BlockSpec((1, N, d_ff), lambda b, i: (b, 0, 0)),
            pl.BlockSpec((1, 1, rb), lambda b, i: (b * nb + i, 0, 0)),
            pl.BlockSpec((1, 1, rb), lambda b, i: (b * nb + i, 0, 0)),
            pl.BlockSpec((1, 1, N), lambda b, i: (b, 0, 0)),
            pl.BlockSpec((1, 1, N), lambda b, i: (b, 0, 0)),
            pl.BlockSpec((1, 1), lambda b, i: (0, 0)),
        ],
        out_specs=pl.BlockSpec((1, rb, N), lambda b, i: (b, i, 0)),
        out_shape=jax.ShapeDtypeStruct((B, N, N), jnp.float32),
        compiler_params=pltpu.CompilerParams(
            dimension_semantics=("parallel", "parallel")
        ),
    )(q, kmat, q, kmat, thr, jcut, thr_col, jcut_col, scale)

    return out


# speedup vs baseline: 1.1599x; 1.1599x over previous
import functools
import math

import jax
import jax.numpy as jnp
from jax.experimental import pallas as pl
from jax.experimental.pallas import tpu as pltpu


def _sortkey(x):
    i = jax.lax.bitcast_convert_type(x, jnp.int32)
    key = i ^ ((i >> 31) & jnp.int32(0x7FFFFFFF))
    return jax.lax.bitcast_convert_type(key, jnp.uint32) ^ jnp.uint32(0x80000000)


def _proj_kernel(x_ref, w1_ref, b1_ref, w2_ref, b2_ref, q_ref, k_ref):
    x = x_ref[0]
    dn = (((1,), (1,)), ((), ()))
    q_ref[0] = (
        jax.lax.dot_general(x, w1_ref[...], dn, preferred_element_type=jnp.float32)
        + b1_ref[...]
    )
    k_ref[0] = (
        jax.lax.dot_general(x, w2_ref[...], dn, preferred_element_type=jnp.float32)
        + b2_ref[...]
    )


def _thresh_kernel(kk, q_ref, kall_ref, t_ref, j_ref):
    q = q_ref[0]
    ka = kall_ref[0]
    dn = (((1,), (1,)), ((), ()))
    s = jax.lax.dot_general(q, ka, dn, preferred_element_type=jnp.float32)
    u = _sortkey(s)
    rb, n = s.shape
    i16 = jnp.int16
    one16 = jnp.ones((), i16)
    zero16 = jnp.zeros((), i16)

    ui = jax.lax.bitcast_convert_type(u, jnp.int32)
    hi = ((ui >> 16) & 0xFFFF) - 32768
    lo = (ui & 0xFFFF) - 32768
    hi16 = hi.astype(i16)
    lo16 = lo.astype(i16)

    def hsum(m16):
        w = m16.shape[1]
        while w > 512:
            w //= 2
            m16 = m16[:, :w] + m16[:, w:]
        return jnp.sum(m16.astype(jnp.int32), axis=1, keepdims=True)

    def count_ge(data, cand16):
        return hsum(jnp.where(data >= cand16, one16, zero16))

    def hi_body(i, t):
        cand = t | (jnp.int32(1) << (jnp.int32(15) - i))
        cnt = count_ge(hi16, (cand - 32768).astype(i16))
        return jnp.where(cnt >= kk, cand, t)

    th = jax.lax.fori_loop(0, 16, hi_body, jnp.zeros((rb, 1), jnp.int32))
    th16 = (th - 32768).astype(i16)

    cnt_gt_hi = hsum(jnp.where(hi16 > th16, one16, zero16))
    band = hi16 == th16
    lob = jnp.where(band, lo16, jnp.int16(-32768))
    need_lo = kk - cnt_gt_hi

    def lo_body(i, t):
        cand = t | (jnp.int32(1) << (jnp.int32(15) - i))
        cnt = count_ge(lob, (cand - 32768).astype(i16))
        return jnp.where(cnt >= need_lo, cand, t)

    tl = jax.lax.fori_loop(0, 16, lo_body, jnp.zeros((rb, 1), jnp.int32))
    tl16 = (tl - 32768).astype(i16)

    cnt_gt = cnt_gt_hi + hsum(jnp.where(lob > tl16, one16, zero16))
    need = kk - cnt_gt
    eq = band & (lo16 == tl16)
    col = jax.lax.broadcasted_iota(i16, (rb, n), 1)

    def j_body(i, carry):
        jlo, jhi = carry
        mid = (jlo + jhi) >> 1
        cnt = hsum(jnp.where(eq & (col <= mid.astype(i16)), one16, zero16))
        p = cnt >= need
        return jnp.where(p, jlo, mid + 1), jnp.where(p, mid, jhi)

    jlo, _ = jax.lax.fori_loop(
        0,
        max(1, (n - 1).bit_length()),
        j_body,
        (jnp.zeros((rb, 1), jnp.int32), jnp.full((rb, 1), n - 1, jnp.int32)),
    )
    t = jax.lax.bitcast_convert_type((th << 16) | tl, jnp.uint32)
    t_ref[0] = t.reshape(1, rb)
    j_ref[0] = jlo.reshape(1, rb)


def _mask_kernel(rb, qi_ref, ki_ref, qall_ref, kall_ref, trow_ref, jrow_ref,
                 tcol_ref, jcol_ref, sc_ref, out_ref):
    qi = qi_ref[0]
    ki = ki_ref[0]
    qa = qall_ref[0]
    ka = kall_ref[0]
    dn = (((1,), (1,)), ((), ()))
    s1 = jax.lax.dot_general(qi, ka, dn, preferred_element_type=jnp.float32)
    s2 = jax.lax.dot_general(ki, qa, dn, preferred_element_type=jnp.float32)
    u1 = _sortkey(s1)
    u2 = _sortkey(s2)
    n = s1.shape[1]

    t1 = trow_ref[0].reshape(rb, 1)
    j1 = jrow_ref[0].reshape(rb, 1)
    t2 = tcol_ref[0]
    j2 = jcol_ref[0]

    col = jax.lax.broadcasted_iota(jnp.int32, (rb, n), 1)
    row = jax.lax.broadcasted_iota(jnp.int32, (rb, n), 0) + pl.program_id(1) * rb

    m1 = (u1 > t1) | ((u1 == t1) & (col <= j1))
    m2 = (u2 > t2) | ((u2 == t2) & (row <= j2))
    out_ref[0] = (m1.astype(jnp.float32) + m2.astype(jnp.float32)) * sc_ref[0, 0]


def kernel(src_temp, W1, b1, W2, b2, k):
    B, N, d_model = src_temp.shape
    d_ff = W1.shape[0]
    kk = d_ff

    rb = 256
    while N % rb:
        rb //= 2
    nb = N // rb

    b1r = b1.reshape(1, d_ff)
    b2r = b2.reshape(1, d_ff)

    q, kmat = pl.pallas_call(
        _proj_kernel,
        grid=(B,),
        in_specs=[
            pl.BlockSpec((1, N, d_model), lambda b: (b, 0, 0)),
            pl.BlockSpec((d_ff, d_model), lambda b: (0, 0)),
            pl.BlockSpec((1, d_ff), lambda b: (0, 0)),
            pl.BlockSpec((d_ff, d_model), lambda b: (0, 0)),
            pl.BlockSpec((1, d_ff), lambda b: (0, 0)),
        ],
        out_specs=[
            pl.BlockSpec((1, N, d_ff), lambda b: (b, 0, 0)),
            pl.BlockSpec((1, N, d_ff), lambda b: (b, 0, 0)),
        ],
        out_shape=[
            jax.ShapeDtypeStruct((B, N, d_ff), jnp.float32),
            jax.ShapeDtypeStruct((B, N, d_ff), jnp.float32),
        ],
        compiler_params=pltpu.CompilerParams(dimension_semantics=("parallel",)),
    )(src_temp, W1, b1r, W2, b2r)

    thr, jcut = pl.pallas_call(
        functools.partial(_thresh_kernel, kk),
        grid=(B, nb),
        in_specs=[
            pl.BlockSpec((1, rb, d_ff), lambda b, i: (b, i, 0)),
            pl.BlockSpec((1, N, d_ff), lambda b, i: (b, 0, 0)),
        ],
        out_specs=[
            pl.BlockSpec((1, 1, rb), lambda b, i: (b * nb + i, 0, 0)),
            pl.BlockSpec((1, 1, rb), lambda b, i: (b * nb + i, 0, 0)),
        ],
        out_shape=[
            jax.ShapeDtypeStruct((B * nb, 1, rb), jnp.uint32),
            jax.ShapeDtypeStruct((B * nb, 1, rb), jnp.int32),
        ],
        compiler_params=pltpu.CompilerParams(
            dimension_semantics=("parallel", "parallel")
        ),
    )(q, kmat)

    thr_col = thr.reshape(B, 1, N)
    jcut_col = jcut.reshape(B, 1, N)
    scale = (jnp.asarray(k, jnp.float32) * (0.5 / d_ff)).reshape(1, 1)

    out = pl.pallas_call(
        functools.partial(_mask_kernel, rb),
        grid=(B, nb),
        in_specs=[
            pl.BlockSpec((1, rb, d_ff), lambda b, i: (b, i, 0)),
            pl.BlockSpec((1, rb, d_ff), lambda b, i: (b, i, 0)),
            pl.BlockSpec((1, N, d_ff), lambda b, i: (b, 0, 0)),
            pl.BlockSpec((1, N, d_ff), lambda b, i: (b, 0, 0)),
            pl.BlockSpec((1, 1, rb), lambda b, i: (b * nb + i, 0, 0)),
            pl.BlockSpec((1, 1, rb), lambda b, i: (b * nb + i, 0, 0)),
            pl.BlockSpec((1, 1, N), lambda b, i: (b, 0, 0)),
            pl.BlockSpec((1, 1, N), lambda b, i: (b, 0, 0)),
            pl.BlockSpec((1, 1), lambda b, i: (0, 0)),
        ],
        out_specs=pl.BlockSpec((1, rb, N), lambda b, i: (b, i, 0)),
        out_shape=jax.ShapeDtypeStruct((B, N, N), jnp.float32),
        compiler_params=pltpu.CompilerParams(
            dimension_semantics=("parallel", "parallel")
        ),
    )(q, kmat, q, kmat, thr, jcut, thr_col, jcut_col, scale)

    return out
